# baseline (speedup 1.0000x reference)
import jax
import jax.numpy as jnp
from jax.experimental import pallas as pl
from jax.experimental.pallas import tpu as pltpu

N = 2048
NINP = 512
NHID = 256
HEADS = 8
NOUT = 256
R = 256
NB = N // R
LOG2E = 1.4426950408889634


def _pre1_kernel(x_ref, w_ref, wa_ref, wh_ref, sd_ref):
    x = x_ref[...]
    for h in range(HEADS):
        wh_ref[h, :, :] = jnp.dot(
            x, w_ref[:, h * NHID:(h + 1) * NHID],
            preferred_element_type=jnp.float32).astype(jnp.bfloat16)
    sd_ref[...] = jnp.dot(x, wa_ref[...], preferred_element_type=jnp.float32).T


def _attn1_kernel(adj_ref, wh_ref, sd_ref, w2_ref, att_ref, wh2_ref):
    i = pl.program_id(0)
    h = pl.program_id(1)
    s = sd_ref[h, pl.ds(i * R, R)]
    d = sd_ref[HEADS + h, :]
    e = s[:, None] + d[None, :]
    e = jnp.maximum(e, 0.2 * e)
    p = jnp.exp2(e) * adj_ref[...]
    r = 1.0 / jnp.sum(p, axis=1, keepdims=True)
    att_ref[0, :, :] = p * r
    o = jnp.dot(p.astype(jnp.bfloat16), wh_ref[h],
                preferred_element_type=jnp.float32) * r
    o = jnp.where(o > 0, o, jnp.exp(jnp.minimum(o, 0.0)) - 1.0)
    part = jnp.dot(o.astype(jnp.bfloat16), w2_ref[h],
                   preferred_element_type=jnp.float32)

    @pl.when(h == 0)
    def _():
        wh2_ref[...] = part

    @pl.when(h > 0)
    def _():
        wh2_ref[...] += part


def _attn2_kernel(adj_ref, wh_ref, a2_ref, att_ref, h2_ref, sd_ref, whb_ref):
    i = pl.program_id(0)

    @pl.when(i == 0)
    def _():
        sd = jnp.dot(wh_ref[...], a2_ref[...],
                     preferred_element_type=jnp.float32)
        sd_ref[...] = sd.T * jnp.float32(LOG2E)
        whb_ref[...] = wh_ref[...].astype(jnp.bfloat16)

    s = sd_ref[0, pl.ds(i * R, R)]
    d = sd_ref[1, :]
    e = s[:, None] + d[None, :]
    e = jnp.maximum(e, 0.2 * e)
    p = jnp.exp2(e) * adj_ref[...]
    r = 1.0 / jnp.sum(p, axis=1, keepdims=True)
    att_ref[...] = p * r
    h2_ref[...] = jnp.dot(p.astype(jnp.bfloat16), whb_ref[...],
                          preferred_element_type=jnp.float32) * r


def kernel(inp, adj, W1, a1_src, a1_dst, W2, a2_src, a2_dst):
    f32 = jnp.float32
    W1h = W1.reshape(NINP, HEADS, NHID)
    wa1 = jnp.concatenate([
        jnp.einsum('ihd,hd->ih', W1h, a1_src),
        jnp.einsum('ihd,hd->ih', W1h, a1_dst),
    ], axis=1) * f32(LOG2E)
    W2r = W2.reshape(HEADS, NHID, NOUT).astype(jnp.bfloat16)
    A2 = jnp.stack([a2_src, a2_dst], axis=1)

    Wh1, sd1 = pl.pallas_call(
        _pre1_kernel,
        grid=(NB,),
        in_specs=[
            pl.BlockSpec((R, NINP), lambda i: (i, 0)),
            pl.BlockSpec((NINP, N), lambda i: (0, 0)),
            pl.BlockSpec((NINP, 2 * HEADS), lambda i: (0, 0)),
        ],
        out_specs=[
            pl.BlockSpec((HEADS, R, NHID), lambda i: (0, i, 0)),
            pl.BlockSpec((2 * HEADS, R), lambda i: (0, i)),
        ],
        out_shape=[
            jax.ShapeDtypeStruct((HEADS, N, NHID), jnp.bfloat16),
            jax.ShapeDtypeStruct((2 * HEADS, N), f32),
        ],
    )(inp, W1, wa1)

    att1, Wh2 = pl.pallas_call(
        _attn1_kernel,
        grid=(NB, HEADS),
        in_specs=[
            pl.BlockSpec((R, N), lambda i, h: (i, 0)),
            pl.BlockSpec((HEADS, N, NHID), lambda i, h: (0, 0, 0)),
            pl.BlockSpec((2 * HEADS, N), lambda i, h: (0, 0)),
            pl.BlockSpec((HEADS, NHID, NOUT), lambda i, h: (0, 0, 0)),
        ],
        out_specs=[
            pl.BlockSpec((1, R, N), lambda i, h: (h, i, 0)),
            pl.BlockSpec((R, NOUT), lambda i, h: (i, 0)),
        ],
        out_shape=[
            jax.ShapeDtypeStruct((HEADS, N, N), f32),
            jax.ShapeDtypeStruct((N, NOUT), f32),
        ],
    )(adj, Wh1, sd1, W2r)

    att2, h2 = pl.pallas_call(
        _attn2_kernel,
        grid=(NB,),
        in_specs=[
            pl.BlockSpec((R, N), lambda i: (i, 0)),
            pl.BlockSpec((N, NOUT), lambda i: (0, 0)),
            pl.BlockSpec((NOUT, 2), lambda i: (0, 0)),
        ],
        out_specs=[
            pl.BlockSpec((R, N), lambda i: (i, 0)),
            pl.BlockSpec((R, NOUT), lambda i: (i, 0)),
        ],
        out_shape=[
            jax.ShapeDtypeStruct((N, N), f32),
            jax.ShapeDtypeStruct((N, NOUT), f32),
        ],
        scratch_shapes=[pltpu.VMEM((2, N), f32),
                        pltpu.VMEM((N, NOUT), jnp.bfloat16)],
    )(adj, Wh2, A2)

    return (h2, att1, att2)


# speedup vs baseline: 1.1867x; 1.1867x over previous
import jax
import jax.numpy as jnp
from jax.experimental import pallas as pl
from jax.experimental.pallas import tpu as pltpu

N = 2048
NINP = 512
NHID = 256
HEADS = 8
NOUT = 256
R = 512
NB = N // R
LOG2E = 1.4426950408889634


def _pre1_kernel(x_ref, w_ref, wa_ref, wh_ref, sd_ref):
    x = x_ref[...]
    for h in range(HEADS):
        wh_ref[h, :, :] = jnp.dot(
            x, w_ref[:, h * NHID:(h + 1) * NHID],
            preferred_element_type=jnp.float32).astype(jnp.bfloat16)
    sd_ref[...] = jnp.dot(x, wa_ref[...], preferred_element_type=jnp.float32).T


def _attn1_kernel(adj_ref, wh_ref, sd_ref, w2_ref, att_ref, wh2_ref):
    i = pl.program_id(0)
    h = pl.program_id(1)
    s = sd_ref[h, pl.ds(i * R, R)]
    d = sd_ref[HEADS + h, :]
    e = s[:, None] + d[None, :]
    e = jnp.maximum(e, 0.2 * e)
    p = jnp.exp2(e) * adj_ref[pl.ds(i * R, R), :]
    r = 1.0 / jnp.sum(p, axis=1, keepdims=True)
    att_ref[0, :, :] = p * r
    o = jnp.dot(p.astype(jnp.bfloat16), wh_ref[h],
                preferred_element_type=jnp.float32) * r
    o = jnp.where(o > 0, o, jnp.exp(jnp.minimum(o, 0.0)) - 1.0)
    part = jnp.dot(o.astype(jnp.bfloat16), w2_ref[h],
                   preferred_element_type=jnp.float32)

    @pl.when(h == 0)
    def _():
        wh2_ref[...] = part

    @pl.when(h > 0)
    def _():
        wh2_ref[...] += part


def _attn2_kernel(adj_ref, wh_ref, a2_ref, att_ref, h2_ref, sd_ref, whb_ref):
    i = pl.program_id(0)

    @pl.when(i == 0)
    def _():
        sd = jnp.dot(wh_ref[...], a2_ref[...],
                     preferred_element_type=jnp.float32)
        sd_ref[...] = sd.T * jnp.float32(LOG2E)
        whb_ref[...] = wh_ref[...].astype(jnp.bfloat16)

    s = sd_ref[0, pl.ds(i * R, R)]
    d = sd_ref[1, :]
    e = s[:, None] + d[None, :]
    e = jnp.maximum(e, 0.2 * e)
    p = jnp.exp2(e) * adj_ref[...]
    r = 1.0 / jnp.sum(p, axis=1, keepdims=True)
    att_ref[...] = p * r
    h2_ref[...] = jnp.dot(p.astype(jnp.bfloat16), whb_ref[...],
                          preferred_element_type=jnp.float32) * r


def kernel(inp, adj, W1, a1_src, a1_dst, W2, a2_src, a2_dst):
    f32 = jnp.float32
    W1h = W1.reshape(NINP, HEADS, NHID)
    wa1 = jnp.concatenate([
        jnp.einsum('ihd,hd->ih', W1h, a1_src),
        jnp.einsum('ihd,hd->ih', W1h, a1_dst),
    ], axis=1) * f32(LOG2E)
    W2r = W2.reshape(HEADS, NHID, NOUT).astype(jnp.bfloat16)
    A2 = jnp.stack([a2_src, a2_dst], axis=1)

    Wh1, sd1 = pl.pallas_call(
        _pre1_kernel,
        grid=(NB,),
        in_specs=[
            pl.BlockSpec((R, NINP), lambda i: (i, 0)),
            pl.BlockSpec((NINP, N), lambda i: (0, 0)),
            pl.BlockSpec((NINP, 2 * HEADS), lambda i: (0, 0)),
        ],
        out_specs=[
            pl.BlockSpec((HEADS, R, NHID), lambda i: (0, i, 0)),
            pl.BlockSpec((2 * HEADS, R), lambda i: (0, i)),
        ],
        out_shape=[
            jax.ShapeDtypeStruct((HEADS, N, NHID), jnp.bfloat16),
            jax.ShapeDtypeStruct((2 * HEADS, N), f32),
        ],
    )(inp, W1, wa1)

    att1, Wh2 = pl.pallas_call(
        _attn1_kernel,
        grid=(NB, HEADS),
        in_specs=[
            pl.BlockSpec((N, N), lambda i, h: (0, 0)),
            pl.BlockSpec((HEADS, N, NHID), lambda i, h: (0, 0, 0)),
            pl.BlockSpec((2 * HEADS, N), lambda i, h: (0, 0)),
            pl.BlockSpec((HEADS, NHID, NOUT), lambda i, h: (0, 0, 0)),
        ],
        out_specs=[
            pl.BlockSpec((1, R, N), lambda i, h: (h, i, 0)),
            pl.BlockSpec((R, NOUT), lambda i, h: (i, 0)),
        ],
        out_shape=[
            jax.ShapeDtypeStruct((HEADS, N, N), f32),
            jax.ShapeDtypeStruct((N, NOUT), f32),
        ],
    )(adj, Wh1, sd1, W2r)

    att2, h2 = pl.pallas_call(
        _attn2_kernel,
        grid=(NB,),
        in_specs=[
            pl.BlockSpec((R, N), lambda i: (i, 0)),
            pl.BlockSpec((N, NOUT), lambda i: (0, 0)),
            pl.BlockSpec((NOUT, 2), lambda i: (0, 0)),
        ],
        out_specs=[
            pl.BlockSpec((R, N), lambda i: (i, 0)),
            pl.BlockSpec((R, NOUT), lambda i: (i, 0)),
        ],
        out_shape=[
            jax.ShapeDtypeStruct((N, N), f32),
            jax.ShapeDtypeStruct((N, NOUT), f32),
        ],
        scratch_shapes=[pltpu.VMEM((2, N), f32),
                        pltpu.VMEM((N, NOUT), jnp.bfloat16)],
    )(adj, Wh2, A2)

    return (h2, att1, att2)


# speedup vs baseline: 1.2403x; 1.0452x over previous
import jax
import jax.numpy as jnp
from jax.experimental import pallas as pl
from jax.experimental.pallas import tpu as pltpu

N = 2048
NINP = 512
NHID = 256
HEADS = 8
NOUT = 256
R = 512
NB = N // R
CC = 512
LOG2E = 1.4426950408889634


def _pre1_kernel(x_ref, w_ref, wa_ref, wh_ref, sd_ref):
    x = x_ref[...]
    for h in range(HEADS):
        wh_ref[h, :, :] = jnp.dot(
            x, w_ref[:, h * NHID:(h + 1) * NHID],
            preferred_element_type=jnp.float32).astype(jnp.bfloat16)
    sd_ref[...] = jnp.dot(x, wa_ref[...], preferred_element_type=jnp.float32).T


def _attn1_kernel(adj_ref, wh_ref, sd_ref, w2_ref, att_ref, wh2_ref):
    i = pl.program_id(0)
    h = pl.program_id(1)
    s = sd_ref[h, pl.ds(i * R, R)]
    sc = s[:, None]
    o = jnp.zeros((R, NHID), jnp.float32)
    tot = jnp.zeros((R, 1), jnp.float32)
    for c in range(N // CC):
        d = sd_ref[HEADS + h, pl.ds(c * CC, CC)]
        e = sc + d[None, :]
        e = jnp.maximum(e, 0.2 * e)
        pc = jnp.exp2(e) * adj_ref[pl.ds(i * R, R), pl.ds(c * CC, CC)]
        att_ref[0, :, pl.ds(c * CC, CC)] = pc
        tot += jnp.sum(pc, axis=1, keepdims=True)
        o += jnp.dot(pc.astype(jnp.bfloat16), wh_ref[h, pl.ds(c * CC, CC), :],
                     preferred_element_type=jnp.float32)
    r = 1.0 / tot
    att_ref[0, :, :] *= r
    o = o * r
    o = jnp.where(o > 0, o, jnp.exp(jnp.minimum(o, 0.0)) - 1.0)
    part = jnp.dot(o.astype(jnp.bfloat16), w2_ref[h],
                   preferred_element_type=jnp.float32)

    @pl.when(h == 0)
    def _():
        wh2_ref[...] = part

    @pl.when(h > 0)
    def _():
        wh2_ref[...] += part


def _attn2_kernel(adj_ref, wh_ref, a2_ref, att_ref, h2_ref, sd_ref, whb_ref):
    i = pl.program_id(0)

    @pl.when(i == 0)
    def _():
        sd = jnp.dot(wh_ref[...], a2_ref[...],
                     preferred_element_type=jnp.float32)
        sd_ref[...] = sd.T * jnp.float32(LOG2E)
        whb_ref[...] = wh_ref[...].astype(jnp.bfloat16)

    s = sd_ref[0, pl.ds(i * R, R)]
    d = sd_ref[1, :]
    e = s[:, None] + d[None, :]
    e = jnp.maximum(e, 0.2 * e)
    p = jnp.exp2(e) * adj_ref[...]
    r = 1.0 / jnp.sum(p, axis=1, keepdims=True)
    att_ref[...] = p * r
    h2_ref[...] = jnp.dot(p.astype(jnp.bfloat16), whb_ref[...],
                          preferred_element_type=jnp.float32) * r


def kernel(inp, adj, W1, a1_src, a1_dst, W2, a2_src, a2_dst):
    f32 = jnp.float32
    W1h = W1.reshape(NINP, HEADS, NHID)
    wa1 = jnp.concatenate([
        jnp.einsum('ihd,hd->ih', W1h, a1_src),
        jnp.einsum('ihd,hd->ih', W1h, a1_dst),
    ], axis=1) * f32(LOG2E)
    W2r = W2.reshape(HEADS, NHID, NOUT).astype(jnp.bfloat16)
    A2 = jnp.stack([a2_src, a2_dst], axis=1)

    Wh1, sd1 = pl.pallas_call(
        _pre1_kernel,
        grid=(NB,),
        in_specs=[
            pl.BlockSpec((R, NINP), lambda i: (i, 0)),
            pl.BlockSpec((NINP, N), lambda i: (0, 0)),
            pl.BlockSpec((NINP, 2 * HEADS), lambda i: (0, 0)),
        ],
        out_specs=[
            pl.BlockSpec((HEADS, R, NHID), lambda i: (0, i, 0)),
            pl.BlockSpec((2 * HEADS, R), lambda i: (0, i)),
        ],
        out_shape=[
            jax.ShapeDtypeStruct((HEADS, N, NHID), jnp.bfloat16),
            jax.ShapeDtypeStruct((2 * HEADS, N), f32),
        ],
    )(inp, W1, wa1)

    att1, Wh2 = pl.pallas_call(
        _attn1_kernel,
        grid=(NB, HEADS),
        in_specs=[
            pl.BlockSpec((N, N), lambda i, h: (0, 0)),
            pl.BlockSpec((HEADS, N, NHID), lambda i, h: (0, 0, 0)),
            pl.BlockSpec((2 * HEADS, N), lambda i, h: (0, 0)),
            pl.BlockSpec((HEADS, NHID, NOUT), lambda i, h: (0, 0, 0)),
        ],
        out_specs=[
            pl.BlockSpec((1, R, N), lambda i, h: (h, i, 0)),
            pl.BlockSpec((R, NOUT), lambda i, h: (i, 0)),
        ],
        out_shape=[
            jax.ShapeDtypeStruct((HEADS, N, N), f32),
            jax.ShapeDtypeStruct((N, NOUT), f32),
        ],
    )(adj, Wh1, sd1, W2r)

    att2, h2 = pl.pallas_call(
        _attn2_kernel,
        grid=(NB,),
        in_specs=[
            pl.BlockSpec((R, N), lambda i: (i, 0)),
            pl.BlockSpec((N, NOUT), lambda i: (0, 0)),
            pl.BlockSpec((NOUT, 2), lambda i: (0, 0)),
        ],
        out_specs=[
            pl.BlockSpec((R, N), lambda i: (i, 0)),
            pl.BlockSpec((R, NOUT), lambda i: (i, 0)),
        ],
        out_shape=[
            jax.ShapeDtypeStruct((N, N), f32),
            jax.ShapeDtypeStruct((N, NOUT), f32),
        ],
        scratch_shapes=[pltpu.VMEM((2, N), f32),
                        pltpu.VMEM((N, NOUT), jnp.bfloat16)],
    )(adj, Wh2, A2)

    return (h2, att1, att2)


# speedup vs baseline: 1.3013x; 1.0492x over previous
import jax
import jax.numpy as jnp
from jax.experimental import pallas as pl
from jax.experimental.pallas import tpu as pltpu

N = 2048
NINP = 512
NHID = 256
HEADS = 8
NOUT = 256
R = 512
NB = N // R
CC = 512
LOG2E = 1.4426950408889634


def _pre1_kernel(x_ref, w_ref, wa_ref, wh_ref, sd_ref):
    x = x_ref[...]
    for h in range(HEADS):
        wh_ref[h, :, :] = jnp.dot(
            x, w_ref[:, h * NHID:(h + 1) * NHID],
            preferred_element_type=jnp.float32).astype(jnp.bfloat16)
    sd_ref[...] = jnp.dot(x, wa_ref[...], preferred_element_type=jnp.float32).T


def _attn12_kernel(adj_ref, wh_ref, sd_ref, w2_ref, a2_ref,
                   att1_ref, att2_ref, h2_ref,
                   wh2_ref, sd2_ref, whb_ref):
    h = pl.program_id(0)
    i = pl.program_id(1)

    @pl.when(h < HEADS)
    def _layer1():
        s = sd_ref[h, pl.ds(i * R, R)]
        sc = s[:, None]
        o = jnp.zeros((R, NHID), jnp.float32)
        tot = jnp.zeros((R, 1), jnp.float32)
        for c in range(N // CC):
            d = sd_ref[HEADS + h, pl.ds(c * CC, CC)]
            e = sc + d[None, :]
            e = jnp.maximum(e, 0.2 * e)
            pc = jnp.exp2(e) * adj_ref[pl.ds(i * R, R), pl.ds(c * CC, CC)]
            att1_ref[0, :, pl.ds(c * CC, CC)] = pc
            tot += jnp.sum(pc, axis=1, keepdims=True)
            o += jnp.dot(pc.astype(jnp.bfloat16),
                         wh_ref[0, pl.ds(c * CC, CC), :],
                         preferred_element_type=jnp.float32)
        r = 1.0 / tot
        att1_ref[0, :, :] *= r
        o = o * r
        o = jnp.where(o > 0, o, jnp.exp(jnp.minimum(o, 0.0)) - 1.0)
        part = jnp.dot(o.astype(jnp.bfloat16), w2_ref[h],
                       preferred_element_type=jnp.float32)

        @pl.when(h == 0)
        def _():
            wh2_ref[pl.ds(i * R, R), :] = part

        @pl.when(h > 0)
        def _():
            wh2_ref[pl.ds(i * R, R), :] += part

    @pl.when(h == HEADS)
    def _layer2():
        @pl.when(i == 0)
        def _():
            sd = jnp.dot(wh2_ref[...], a2_ref[...],
                         preferred_element_type=jnp.float32)
            sd2_ref[...] = sd.T * jnp.float32(LOG2E)
            whb_ref[...] = wh2_ref[...].astype(jnp.bfloat16)

        s = sd2_ref[0, pl.ds(i * R, R)]
        sc = s[:, None]
        o = jnp.zeros((R, NOUT), jnp.float32)
        tot = jnp.zeros((R, 1), jnp.float32)
        for c in range(N // CC):
            d = sd2_ref[1, pl.ds(c * CC, CC)]
            e = sc + d[None, :]
            e = jnp.maximum(e, 0.2 * e)
            pc = jnp.exp2(e) * adj_ref[pl.ds(i * R, R), pl.ds(c * CC, CC)]
            att2_ref[:, pl.ds(c * CC, CC)] = pc
            tot += jnp.sum(pc, axis=1, keepdims=True)
            o += jnp.dot(pc.astype(jnp.bfloat16),
                         whb_ref[pl.ds(c * CC, CC), :],
                         preferred_element_type=jnp.float32)
        r = 1.0 / tot
        att2_ref[...] *= r
        h2_ref[...] = o * r


def kernel(inp, adj, W1, a1_src, a1_dst, W2, a2_src, a2_dst):
    f32 = jnp.float32
    W1h = W1.reshape(NINP, HEADS, NHID)
    wa1 = jnp.concatenate([
        jnp.einsum('ihd,hd->ih', W1h, a1_src),
        jnp.einsum('ihd,hd->ih', W1h, a1_dst),
    ], axis=1) * f32(LOG2E)
    W2r = W2.reshape(HEADS, NHID, NOUT).astype(jnp.bfloat16)
    A2 = jnp.stack([a2_src, a2_dst], axis=1)

    Wh1, sd1 = pl.pallas_call(
        _pre1_kernel,
        grid=(NB,),
        in_specs=[
            pl.BlockSpec((R, NINP), lambda i: (i, 0)),
            pl.BlockSpec((NINP, N), lambda i: (0, 0)),
            pl.BlockSpec((NINP, 2 * HEADS), lambda i: (0, 0)),
        ],
        out_specs=[
            pl.BlockSpec((HEADS, R, NHID), lambda i: (0, i, 0)),
            pl.BlockSpec((2 * HEADS, R), lambda i: (0, i)),
        ],
        out_shape=[
            jax.ShapeDtypeStruct((HEADS, N, NHID), jnp.bfloat16),
            jax.ShapeDtypeStruct((2 * HEADS, N), f32),
        ],
    )(inp, W1, wa1)

    h7 = HEADS - 1

    att1, att2, h2 = pl.pallas_call(
        _attn12_kernel,
        grid=(HEADS + 1, NB),
        in_specs=[
            pl.BlockSpec((N, N), lambda h, i: (0, 0)),
            pl.BlockSpec((1, N, NHID),
                         lambda h, i: (jnp.minimum(h, h7), 0, 0)),
            pl.BlockSpec((2 * HEADS, N), lambda h, i: (0, 0)),
            pl.BlockSpec((HEADS, NHID, NOUT), lambda h, i: (0, 0, 0)),
            pl.BlockSpec((NOUT, 2), lambda h, i: (0, 0)),
        ],
        out_specs=[
            pl.BlockSpec((1, R, N),
                         lambda h, i: (jnp.minimum(h, h7),
                                       jnp.where(h < HEADS, i, NB - 1), 0)),
            pl.BlockSpec((R, N),
                         lambda h, i: (jnp.where(h < HEADS, 0, i), 0)),
            pl.BlockSpec((R, NOUT),
                         lambda h, i: (jnp.where(h < HEADS, 0, i), 0)),
        ],
        out_shape=[
            jax.ShapeDtypeStruct((HEADS, N, N), f32),
            jax.ShapeDtypeStruct((N, N), f32),
            jax.ShapeDtypeStruct((N, NOUT), f32),
        ],
        scratch_shapes=[
            pltpu.VMEM((N, NOUT), f32),
            pltpu.VMEM((2, N), f32),
            pltpu.VMEM((N, NOUT), jnp.bfloat16),
        ],
    )(adj, Wh1, sd1, W2r, A2)

    return (h2, att1, att2)
